# Initial kernel scaffold; baseline (speedup 1.0000x reference)
import jax
import jax.numpy as jnp
from jax.experimental import pallas as pl


def kernel(node_representations, edges, edge_weights, gamma1, beta1, mean1, var1, W1, b1, gamma2, beta2, mean2, var2, W2, b2):
    raise NotImplementedError("write your pallas kernel here")


# speedup vs baseline: 3.6901x; 3.6901x over previous
import functools

import jax
import jax.numpy as jnp
from jax import lax
from jax.experimental import pallas as pl
from jax.experimental.pallas import tpu as pltpu
from jax.experimental.pallas import tpu_sc as plsc

N = 10000
E = 320000
D = 128
H = 128
ROWW = 144
CH = 128
NC = 2
NS = 16
NW = NC * NS
NCHUNK = E // CH
FULL = NCHUNK // NW
TAIL = NCHUNK - FULL * NW
RPS = N // NS
ZR = 125

_BLK = 1000


def _gelu(x):
    return 0.5 * x * (1.0 + lax.erf(x * jnp.float32(0.7071067811865476)))


def _prep_body(x_ref, w_ref, b_ref, a_ref, c_ref, out_ref):
    xb = x_ref[...] * a_ref[...] + c_ref[...]
    h = jnp.dot(xb, w_ref[...], preferred_element_type=jnp.float32) + b_ref[...]
    out_ref[...] = _gelu(h)


def _prepare(x, W1, b1, a1, c1):
    return pl.pallas_call(
        _prep_body,
        grid=(N // _BLK,),
        in_specs=[
            pl.BlockSpec((_BLK, D), lambda i: (i, 0)),
            pl.BlockSpec((D, H), lambda i: (0, 0)),
            pl.BlockSpec((1, H), lambda i: (0, 0)),
            pl.BlockSpec((1, D), lambda i: (0, 0)),
            pl.BlockSpec((1, D), lambda i: (0, 0)),
        ],
        out_specs=pl.BlockSpec((_BLK, H), lambda i: (i, 0)),
        out_shape=jax.ShapeDtypeStruct((N, H), jnp.float32),
    )(x, W1, b1, a1, c1)


def _upd_body(x_ref, p_ref, wa_ref, wb_ref, b_ref, ax_ref, cx_ref,
              ag_ref, cg_ref, out_ref):
    s = p_ref[0] + p_ref[1]
    seg = s[:, :H]
    cnt = s[:, H:H + 1]
    agg = jnp.where(cnt > 0.0, seg / jnp.maximum(cnt, 1.0), 0.0)
    xb = x_ref[...] * ax_ref[...] + cx_ref[...]
    gb = agg * ag_ref[...] + cg_ref[...]
    h = (jnp.dot(xb, wa_ref[...], preferred_element_type=jnp.float32)
         + jnp.dot(gb, wb_ref[...], preferred_element_type=jnp.float32)
         + b_ref[...])
    out_ref[...] = _gelu(h)


def _update(x, part, W2a, W2b, b2, a2x, c2x, a2g, c2g):
    return pl.pallas_call(
        _upd_body,
        grid=(N // _BLK,),
        in_specs=[
            pl.BlockSpec((_BLK, D), lambda i: (i, 0)),
            pl.BlockSpec((NC, _BLK, ROWW), lambda i: (0, i, 0)),
            pl.BlockSpec((D, H), lambda i: (0, 0)),
            pl.BlockSpec((H, H), lambda i: (0, 0)),
            pl.BlockSpec((1, H), lambda i: (0, 0)),
            pl.BlockSpec((1, D), lambda i: (0, 0)),
            pl.BlockSpec((1, D), lambda i: (0, 0)),
            pl.BlockSpec((1, H), lambda i: (0, 0)),
            pl.BlockSpec((1, H), lambda i: (0, 0)),
        ],
        out_specs=pl.BlockSpec((_BLK, H), lambda i: (i, 0)),
        out_shape=jax.ShapeDtypeStruct((N, H), jnp.float32),
    )(x, part, W2a, W2b, b2, a2x, c2x, a2g, c2g)


def _sc_body(p_hbm, edges_hbm, w_hbm, out_hbm,
             src_v, dst_v, w_v, rows_v, st_v, acc_sh, sem):
    cid = lax.axis_index("c")
    sid = lax.axis_index("s")
    wid = sid * NC + cid

    lane = lax.broadcasted_iota(jnp.int32, (16,), 0)
    one_hot = jnp.where(lane == 0, 1.0, 0.0).astype(jnp.float32)
    zero16 = jnp.zeros((16,), jnp.float32)

    def _zrow(r, _):
        for c in range(ROWW // 16):
            st_v[r, pl.ds(c * 16, 16)] = zero16
        return 0
    lax.fori_loop(0, ZR, _zrow, 0)
    for j in range(RPS // ZR):
        pltpu.sync_copy(st_v.at[pl.ds(0, ZR)],
                        acc_sh.at[pl.ds(sid * RPS + j * ZR, ZR)])
    plsc.subcore_barrier()

    def _chunk(c):
        base = c * CH
        pltpu.sync_copy(edges_hbm.at[1, pl.ds(base, CH)], src_v)
        pltpu.sync_copy(edges_hbm.at[0, pl.ds(base, CH)], dst_v)
        pltpu.sync_copy(w_hbm.at[pl.ds(base, CH)], w_v)
        pltpu.async_copy(p_hbm.at[src_v], rows_v, sem).wait()

        def _grp(g, _):
            wv16 = w_v[pl.ds(g * 16, 16)]
            for j in range(16):
                r = g * 16 + j
                wb = jnp.full((16,), wv16[j], jnp.float32)
                for cc in range(H // 16):
                    st_v[r, pl.ds(cc * 16, 16)] = (
                        rows_v[r, pl.ds(cc * 16, 16)] * wb)
                st_v[r, pl.ds(H, 16)] = one_hot
            return 0
        lax.fori_loop(0, CH // 16, _grp, 0)
        pltpu.sync_copy(st_v, acc_sh.at[dst_v], add=True)

    def _loop(i, _):
        _chunk(wid + i * NW)
        return 0
    lax.fori_loop(0, FULL, _loop, 0)

    @pl.when(wid < TAIL)
    def _():
        _chunk(FULL * NW + wid)

    plsc.subcore_barrier()

    pltpu.sync_copy(acc_sh.at[pl.ds(sid * RPS, RPS)],
                    out_hbm.at[cid, pl.ds(sid * RPS, RPS)])


@functools.lru_cache(maxsize=1)
def _sc_aggregate_fn():
    return pl.kernel(
        _sc_body,
        out_type=jax.ShapeDtypeStruct((NC, N, ROWW), jnp.float32),
        mesh=plsc.VectorSubcoreMesh(core_axis_name="c", subcore_axis_name="s",
                                    num_cores=NC, num_subcores=NS),
        scratch_types=[
            pltpu.VMEM((CH,), jnp.int32),
            pltpu.VMEM((CH,), jnp.int32),
            pltpu.VMEM((CH,), jnp.float32),
            pltpu.VMEM((CH, H), jnp.float32),
            pltpu.VMEM((CH, ROWW), jnp.float32),
            pltpu.VMEM_SHARED((N, ROWW), jnp.float32),
            pltpu.SemaphoreType.DMA,
        ],
        compiler_params=pltpu.CompilerParams(use_tc_tiling_on_sc=False),
    )


def _sc_aggregate(P, edges, edge_weights):
    return _sc_aggregate_fn()(P, edges, edge_weights)


def kernel(node_representations, edges, edge_weights,
           gamma1, beta1, mean1, var1, W1, b1,
           gamma2, beta2, mean2, var2, W2, b2):
    eps = jnp.float32(1e-3)
    a1 = (gamma1 * lax.rsqrt(var1 + eps)).reshape(1, D)
    c1 = (beta1 - mean1 * gamma1 * lax.rsqrt(var1 + eps)).reshape(1, D)
    a2 = gamma2 * lax.rsqrt(var2 + eps)
    c2 = beta2 - mean2 * gamma2 * lax.rsqrt(var2 + eps)

    P = _prepare(node_representations, W1, b1.reshape(1, H), a1, c1)
    part = _sc_aggregate(P, edges, edge_weights)
    out = _update(node_representations, part, W2[:D], W2[D:],
                  b2.reshape(1, H),
                  a2[:D].reshape(1, D), c2[:D].reshape(1, D),
                  a2[D:].reshape(1, H), c2[D:].reshape(1, H))
    return out


# speedup vs baseline: 11.1261x; 3.0151x over previous
import functools

import jax
import jax.numpy as jnp
from jax import lax
from jax.experimental import pallas as pl
from jax.experimental.pallas import tpu as pltpu
from jax.experimental.pallas import tpu_sc as plsc

N = 10000
E = 320000
D = 128
H = 128
ROWW = 144
CH = 128
NC = 2
NS = 16
NW = NC * NS
NCHUNK = E // CH
FULL = NCHUNK // NW
TAIL = NCHUNK - FULL * NW
RPS = N // NS
BCH = 6
NB = FULL // BCH
BE = BCH * CH

_BLK = 1000


def _gelu(x):
    return 0.5 * x * (1.0 + lax.erf(x * jnp.float32(0.7071067811865476)))


def _prep_body(x_ref, w_ref, b_ref, a_ref, c_ref, out_ref):
    xb = x_ref[...] * a_ref[...] + c_ref[...]
    h = jnp.dot(xb, w_ref[...], preferred_element_type=jnp.float32) + b_ref[...]
    out_ref[:, :H] = _gelu(h)
    lane = lax.broadcasted_iota(jnp.int32, (_BLK, ROWW - H), 1)
    out_ref[:, H:] = jnp.where(lane == 0, 1.0, 0.0).astype(jnp.float32)


def _prepare(x, W1, b1, a1, c1):
    return pl.pallas_call(
        _prep_body,
        grid=(N // _BLK,),
        in_specs=[
            pl.BlockSpec((_BLK, D), lambda i: (i, 0)),
            pl.BlockSpec((D, H), lambda i: (0, 0)),
            pl.BlockSpec((1, H), lambda i: (0, 0)),
            pl.BlockSpec((1, D), lambda i: (0, 0)),
            pl.BlockSpec((1, D), lambda i: (0, 0)),
        ],
        out_specs=pl.BlockSpec((_BLK, ROWW), lambda i: (i, 0)),
        out_shape=jax.ShapeDtypeStruct((N, ROWW), jnp.float32),
    )(x, W1, b1, a1, c1)


def _upd_body(x_ref, p_ref, wa_ref, wb_ref, b_ref, ax_ref, cx_ref,
              ag_ref, cg_ref, out_ref):
    s = p_ref[0] + p_ref[1]
    seg = s[:, :H]
    cnt = s[:, H:H + 1]
    agg = jnp.where(cnt > 0.0, seg / jnp.maximum(cnt, 1.0), 0.0)
    xb = x_ref[...] * ax_ref[...] + cx_ref[...]
    gb = agg * ag_ref[...] + cg_ref[...]
    h = (jnp.dot(xb, wa_ref[...], preferred_element_type=jnp.float32)
         + jnp.dot(gb, wb_ref[...], preferred_element_type=jnp.float32)
         + b_ref[...])
    out_ref[...] = _gelu(h)


def _update(x, part, W2a, W2b, b2, a2x, c2x, a2g, c2g):
    return pl.pallas_call(
        _upd_body,
        grid=(N // _BLK,),
        in_specs=[
            pl.BlockSpec((_BLK, D), lambda i: (i, 0)),
            pl.BlockSpec((NC, _BLK, ROWW), lambda i: (0, i, 0)),
            pl.BlockSpec((D, H), lambda i: (0, 0)),
            pl.BlockSpec((H, H), lambda i: (0, 0)),
            pl.BlockSpec((1, H), lambda i: (0, 0)),
            pl.BlockSpec((1, D), lambda i: (0, 0)),
            pl.BlockSpec((1, D), lambda i: (0, 0)),
            pl.BlockSpec((1, H), lambda i: (0, 0)),
            pl.BlockSpec((1, H), lambda i: (0, 0)),
        ],
        out_specs=pl.BlockSpec((_BLK, H), lambda i: (i, 0)),
        out_shape=jax.ShapeDtypeStruct((N, H), jnp.float32),
    )(x, part, W2a, W2b, b2, a2x, c2x, a2g, c2g)


def _sc_body(p_hbm, src_hbm, dst2d_hbm, w16_hbm, out_hbm,
             bs_a, bw_a, bd_a, bs_b, bw_b, bd_b, rows0, rows1, acc_sh,
             sem_g0, sem_g1, sem_s0, sem_s1, sem_ba, sem_bb):
    cid = lax.axis_index("c")
    sid = lax.axis_index("s")
    wid = sid * NC + cid

    brow = wid * FULL + jnp.minimum(wid, TAIL)
    nkw = FULL + jnp.where(wid < TAIL, 1, 0)

    zero16 = jnp.zeros((16,), jnp.float32)

    def _bload(m, bs, bw, bd, sem):
        row0 = brow + m * BCH
        e0 = row0 * CH
        pltpu.async_copy(src_hbm.at[pl.ds(e0, BE)], bs, sem)
        pltpu.async_copy(w16_hbm.at[pl.ds(e0, BE)], bw, sem)
        pltpu.async_copy(dst2d_hbm.at[pl.ds(row0, BCH)], bd, sem)

    def _bwait(bs, bw, bd, sem):
        pltpu.make_async_copy(src_hbm.at[pl.ds(0, BE)], bs, sem).wait()
        pltpu.make_async_copy(w16_hbm.at[pl.ds(0, BE)], bw, sem).wait()
        pltpu.make_async_copy(dst2d_hbm.at[pl.ds(0, BCH)], bd, sem).wait()

    def _gather(bs, cb, rows, sem):
        pltpu.async_copy(p_hbm.at[bs.at[pl.ds(cb * CH, CH)]], rows, sem)

    def _gwait(rows, sem):
        pltpu.make_async_copy(
            p_hbm.at[bs_a.at[pl.ds(0, CH)]], rows, sem).wait()

    def _scatter(rows, bd, cb, sem):
        pltpu.async_copy(rows, acc_sh.at[bd.at[cb]], sem, add=True)

    def _swait(rows, sem):
        pltpu.make_async_copy(rows, acc_sh.at[bd_a.at[0]], sem).wait()

    def _scale(rows, bw, cb):
        def _row(r, _):
            g32 = (r // 32) * 32
            w32 = bw[pl.ds(cb * CH + g32, 32)]
            we, wo = plsc.unpack(w32, format=plsc.PackFormat.INTERLEAVED)
            wpair = jnp.where(r % 2 == 0, we, wo)
            j2 = (r % 32) // 2
            dnums = lax.GatherDimensionNumbers(
                offset_dims=(), collapsed_slice_dims=(0,),
                start_index_map=(0,))
            wb = lax.gather(wpair, jnp.full((16, 1), j2, jnp.int32),
                            dnums, (1,),
                            mode=lax.GatherScatterMode.PROMISE_IN_BOUNDS)
            for cc in range(H // 16):
                rows[r, pl.ds(cc * 16, 16)] = (
                    rows[r, pl.ds(cc * 16, 16)] * wb)
            return 0
        lax.fori_loop(0, CH, _row, 0)

    def _pbatch(m, bs, bw, bd, os_, ow, od, sem_o, first):
        for cb in range(BCH):
            p = cb % 2
            rows = rows0 if p == 0 else rows1
            semg = sem_g0 if p == 0 else sem_g1
            sems = sem_s0 if p == 0 else sem_s1

            if cb == 2:
                @pl.when((m + 1) * BCH < nkw)
                def _():
                    _bload(m + 1, os_, ow, od, sem_o)

            _gwait(rows, semg)
            if not (first and cb < 2):
                _swait(rows, sems)
            _scale(rows, bw, cb)
            _scatter(rows, bd, cb, sems)

            if cb < BCH - 2:
                _gather(bs, cb + 2, rows, semg)
            elif cb == BCH - 2:
                @pl.when((m + 1) * BCH < nkw)
                def _():
                    _bwait(os_, ow, od, sem_o)
                    _gather(os_, 0, rows0, sem_g0)
            else:
                @pl.when((m + 1) * BCH + 1 < nkw)
                def _():
                    _gather(os_, 1, rows1, sem_g1)

    def _zrow(r, _):
        for c in range(ROWW // 16):
            rows0[r, pl.ds(c * 16, 16)] = zero16
        return 0
    lax.fori_loop(0, CH, _zrow, 0)
    for off, cnt in ((0, CH), (CH, CH), (2 * CH, CH), (3 * CH, CH),
                     (4 * CH, RPS - 4 * CH)):
        pltpu.sync_copy(rows0.at[pl.ds(0, cnt)],
                        acc_sh.at[pl.ds(sid * RPS + off, cnt)])
    plsc.subcore_barrier()

    _bload(0, bs_a, bw_a, bd_a, sem_ba)
    _bwait(bs_a, bw_a, bd_a, sem_ba)
    _gather(bs_a, 0, rows0, sem_g0)
    _gather(bs_a, 1, rows1, sem_g1)

    _pbatch(0, bs_a, bw_a, bd_a, bs_b, bw_b, bd_b, sem_bb, True)

    def _pair(i, _):
        _pbatch(2 * i + 1, bs_b, bw_b, bd_b, bs_a, bw_a, bd_a, sem_ba, False)
        _pbatch(2 * i + 2, bs_a, bw_a, bd_a, bs_b, bw_b, bd_b, sem_bb, False)
        return 0
    lax.fori_loop(0, (NB - 1) // 2, _pair, 0)

    @pl.when(wid < TAIL)
    def _():
        _gwait(rows0, sem_g0)
        _swait(rows0, sem_s0)
        _scale(rows0, bw_b, 0)
        _scatter(rows0, bd_b, 0, sem_s0)

    _swait(rows0, sem_s0)
    _swait(rows1, sem_s1)

    plsc.subcore_barrier()

    pltpu.sync_copy(acc_sh.at[pl.ds(sid * RPS, RPS)],
                    out_hbm.at[cid, pl.ds(sid * RPS, RPS)])


@functools.lru_cache(maxsize=1)
def _sc_aggregate_fn():
    return pl.kernel(
        _sc_body,
        out_type=jax.ShapeDtypeStruct((NC, N, ROWW), jnp.float32),
        mesh=plsc.VectorSubcoreMesh(core_axis_name="c", subcore_axis_name="s",
                                    num_cores=NC, num_subcores=NS),
        scratch_types=[
            pltpu.VMEM((BE,), jnp.int32),
            pltpu.VMEM((BE,), jnp.bfloat16),
            pltpu.VMEM((BCH, CH), jnp.int32),
            pltpu.VMEM((BE,), jnp.int32),
            pltpu.VMEM((BE,), jnp.bfloat16),
            pltpu.VMEM((BCH, CH), jnp.int32),
            pltpu.VMEM((CH, ROWW), jnp.float32),
            pltpu.VMEM((CH, ROWW), jnp.float32),
            pltpu.VMEM_SHARED((N, ROWW), jnp.float32),
            pltpu.SemaphoreType.DMA,
            pltpu.SemaphoreType.DMA,
            pltpu.SemaphoreType.DMA,
            pltpu.SemaphoreType.DMA,
            pltpu.SemaphoreType.DMA,
            pltpu.SemaphoreType.DMA,
        ],
        compiler_params=pltpu.CompilerParams(use_tc_tiling_on_sc=False,
                                             needs_layout_passes=False),
    )


def _sc_aggregate(P, edges, edge_weights):
    dst2d = edges[0].reshape(NCHUNK, CH)
    w16 = edge_weights.astype(jnp.bfloat16)
    return _sc_aggregate_fn()(P, edges[1], dst2d, w16)


def kernel(node_representations, edges, edge_weights,
           gamma1, beta1, mean1, var1, W1, b1,
           gamma2, beta2, mean2, var2, W2, b2):
    eps = jnp.float32(1e-3)
    a1 = (gamma1 * lax.rsqrt(var1 + eps)).reshape(1, D)
    c1 = (beta1 - mean1 * gamma1 * lax.rsqrt(var1 + eps)).reshape(1, D)
    a2 = gamma2 * lax.rsqrt(var2 + eps)
    c2 = beta2 - mean2 * gamma2 * lax.rsqrt(var2 + eps)

    P = _prepare(node_representations, W1, b1.reshape(1, H), a1, c1)
    part = _sc_aggregate(P, edges, edge_weights)
    out = _update(node_representations, part, W2[:D], W2[D:],
                  b2.reshape(1, H),
                  a2[:D].reshape(1, D), c2[:D].reshape(1, D),
                  a2[D:].reshape(1, H), c2[D:].reshape(1, H))
    return out
